# baseline (speedup 1.0000x reference)
import functools

import jax
import jax.numpy as jnp
from jax import lax
from jax.experimental import pallas as pl
from jax.experimental.pallas import tpu as pltpu
from jax.experimental.pallas import tpu_sc as plsc

E = 320000
H = 128
NW = 32
EPW = E // NW
C = 80
NCH = EPW // C
NHEX = 20
G = C // 16

_mesh = plsc.VectorSubcoreMesh(core_axis_name="c", subcore_axis_name="s")

_slot_types = [
    pltpu.VMEM((C, H), jnp.float32),
    pltpu.SemaphoreType.DMA,
]


@functools.partial(
    pl.kernel,
    out_type=jax.ShapeDtypeStruct((E,), jnp.float32),
    mesh=_mesh,
    compiler_params=pltpu.CompilerParams(needs_layout_passes=False),
    scratch_types=[
        pltpu.VMEM((EPW,), jnp.int32),
        pltpu.VMEM((EPW,), jnp.int32),
        pltpu.VMEM((EPW,), jnp.int32),
        pltpu.VMEM((EPW,), jnp.float32),
        pltpu.VMEM_SHARED((500, H), jnp.float32),
    ] + _slot_types * 6,
)
def _transe(z_h, zn_h, src_h, dst_h, et_h, rel_h, out_h, si, di, ti, o, rel_sp,
            *scratch):
    slots = tuple(scratch[2 * k:2 * k + 2] for k in range(6))
    sid = lax.axis_index("s")
    wid = sid * 2 + lax.axis_index("c")
    base = wid * EPW

    @pl.when(sid == 0)
    def _():
        pltpu.sync_copy(rel_h, rel_sp)

    pltpu.sync_copy(src_h.at[pl.ds(base, EPW)], si)
    pltpu.sync_copy(dst_h.at[pl.ds(base, EPW)], di)
    pltpu.sync_copy(et_h.at[pl.ds(base, EPW)], ti)
    plsc.subcore_barrier()

    def fire1(ci, s):
        d, sem = s
        pltpu.make_async_copy(z_h.at[si.at[pl.ds(ci * C, C)]], d, sem).start()

    def fire2(ci, s):
        d, sem = s
        pltpu.make_async_copy(z_h.at[si.at[pl.ds(ci * C, C)]], d, sem).wait()
        pltpu.async_copy(rel_sp.at[ti.at[pl.ds(ci * C, C)]], d, sem, add=True)

    def fire3(ci, s):
        d, sem = s
        pltpu.make_async_copy(rel_sp.at[ti.at[pl.ds(ci * C, C)]], d, sem).wait()
        pltpu.async_copy(zn_h.at[di.at[pl.ds(ci * C, C)]], d, sem, add=True)

    def finish(ci, s):
        d, sem = s
        pltpu.make_async_copy(zn_h.at[di.at[pl.ds(ci * C, C)]], d, sem).wait()

        def group(g, carry):
            lane = lax.iota(jnp.int32, 16)
            rows = g * 16 + lane
            FB = 32

            def fblock(fb, acc):
                for fo in range(FB):
                    fv = (lane + (fb * FB + fo)) & (H - 1)
                    vd = plsc.load_gather(d, [rows, fv])
                    acc = acc + vd * vd
                return acc

            acc = lax.fori_loop(0, H // FB, fblock, jnp.zeros((16,), jnp.float32))
            ibits = plsc.bitcast(acc, jnp.int32)
            magic = jnp.full((16,), 0x5F3759DF, jnp.int32)
            y = plsc.bitcast(magic - (ibits >> 1), jnp.float32)
            for _ in range(3):
                y = y * (1.5 - 0.5 * acc * y * y)
            res = jnp.where(acc > 0.0, -(acc * y), 0.0)
            o[pl.ds(ci * C + g * 16, 16)] = res
            return carry

        lax.fori_loop(0, G, group, 0)

    for c in range(5):
        fire1(c, slots[c])
    for c in range(3):
        fire2(c, slots[c])
    fire3(0, slots[0])

    def hexad(j, carry):
        c0 = j * 6
        for k in range(6):
            ci = c0 + k
            fire1(ci + 5, slots[(k + 5) % 6])
            fire2(ci + 3, slots[(k + 3) % 6])
            fire3(ci + 1, slots[(k + 1) % 6])
            finish(ci, slots[k])
        return carry

    lax.fori_loop(0, NHEX, hexad, 0)
    for ci in range(NHEX * 6, NCH):
        if ci + 5 < NCH:
            fire1(ci + 5, slots[(ci + 5) % 6])
        if ci + 3 < NCH:
            fire2(ci + 3, slots[(ci + 3) % 6])
        if ci + 1 < NCH:
            fire3(ci + 1, slots[(ci + 1) % 6])
        finish(ci, slots[ci % 6])

    pltpu.sync_copy(o, out_h.at[pl.ds(base, EPW)])


def kernel(z, edge_index, edge_type, rel_emb):
    src = edge_index[0].astype(jnp.int32)
    dst = edge_index[1].astype(jnp.int32)
    et = edge_type.astype(jnp.int32)
    return _transe(z, -z, src, dst, et, rel_emb)


# speedup vs baseline: 1.0791x; 1.0791x over previous
import functools

import jax
import jax.numpy as jnp
from jax import lax
from jax.experimental import pallas as pl
from jax.experimental.pallas import tpu as pltpu
from jax.experimental.pallas import tpu_sc as plsc

E = 320000
H = 128
NW = 32
EPW = E // NW
C = 128
NCH = EPW // C
CT = EPW - NCH * C
NQUAD = (NCH - 6) // 4
G = C // 16

_mesh = plsc.VectorSubcoreMesh(core_axis_name="c", subcore_axis_name="s")

_slot_types = [
    pltpu.VMEM((C, H), jnp.float32),
    pltpu.SemaphoreType.DMA,
]


@functools.partial(
    pl.kernel,
    out_type=jax.ShapeDtypeStruct((E,), jnp.float32),
    mesh=_mesh,
    compiler_params=pltpu.CompilerParams(needs_layout_passes=False),
    scratch_types=[
        pltpu.VMEM((EPW,), jnp.int32),
        pltpu.VMEM((EPW,), jnp.int32),
        pltpu.VMEM((EPW,), jnp.int32),
        pltpu.VMEM((EPW,), jnp.float32),
        pltpu.VMEM_SHARED((500, H), jnp.float32),
    ] + _slot_types + _slot_types + _slot_types + _slot_types + [
        pltpu.VMEM((CT, H), jnp.float32),
        pltpu.SemaphoreType.DMA,
    ],
)
def _transe(z_h, zn_h, src_h, dst_h, et_h, rel_h, out_h, si, di, ti, o, rel_sp,
            *scratch):
    slots = tuple(scratch[2 * k:2 * k + 2] for k in range(5))
    sid = lax.axis_index("s")
    wid = sid * 2 + lax.axis_index("c")
    base = wid * EPW

    @pl.when(sid == 0)
    def _():
        pltpu.sync_copy(rel_h, rel_sp)

    pltpu.sync_copy(src_h.at[pl.ds(base, EPW)], si)
    pltpu.sync_copy(dst_h.at[pl.ds(base, EPW)], di)
    pltpu.sync_copy(et_h.at[pl.ds(base, EPW)], ti)
    plsc.subcore_barrier()

    def fire1(ci, s):
        d, sem = s
        pltpu.make_async_copy(z_h.at[si.at[pl.ds(ci * C, C)]], d, sem).start()

    def fire2(ci, s):
        d, sem = s
        pltpu.make_async_copy(z_h.at[si.at[pl.ds(ci * C, C)]], d, sem).wait()
        pltpu.async_copy(rel_sp.at[ti.at[pl.ds(ci * C, C)]], d, sem, add=True)

    def fire3(ci, s):
        d, sem = s
        pltpu.make_async_copy(rel_sp.at[ti.at[pl.ds(ci * C, C)]], d, sem).wait()
        pltpu.async_copy(zn_h.at[di.at[pl.ds(ci * C, C)]], d, sem, add=True)

    def finish(ci, s):
        d, sem = s
        pltpu.make_async_copy(zn_h.at[di.at[pl.ds(ci * C, C)]], d, sem).wait()

        def group(g, carry):
            lane = lax.iota(jnp.int32, 16)
            rows = g * 16 + lane
            FB = 32

            def fblock(fb, acc):
                for fo in range(FB):
                    fv = (lane + (fb * FB + fo)) & (H - 1)
                    vd = plsc.load_gather(d, [rows, fv])
                    acc = acc + vd * vd
                return acc

            acc = lax.fori_loop(0, H // FB, fblock, jnp.zeros((16,), jnp.float32))
            ibits = plsc.bitcast(acc, jnp.int32)
            magic = jnp.full((16,), 0x5F3759DF, jnp.int32)
            y = plsc.bitcast(magic - (ibits >> 1), jnp.float32)
            for _ in range(3):
                y = y * (1.5 - 0.5 * acc * y * y)
            res = jnp.where(acc > 0.0, -(acc * y), 0.0)
            o[pl.ds(ci * C + g * 16, 16)] = res
            return carry

        lax.fori_loop(0, G, group, 0)

    fire1(0, slots[0])
    fire2(0, slots[0])
    fire1(1, slots[1])
    fire3(0, slots[0])
    fire2(1, slots[1])
    fire1(2, slots[2])

    def quad(j, carry):
        c0 = j * 4
        for k in range(4):
            ci = c0 + k
            fire1(ci + 3, slots[(k + 3) % 4])
            fire2(ci + 2, slots[(k + 2) % 4])
            fire3(ci + 1, slots[(k + 1) % 4])
            finish(ci, slots[k])
        return carry

    lax.fori_loop(0, NQUAD, quad, 0)
    for ci in range(NQUAD * 4, NCH):
        if ci + 3 < NCH:
            fire1(ci + 3, slots[(ci + 3) % 4])
        if ci + 2 < NCH:
            fire2(ci + 2, slots[(ci + 2) % 4])
        if ci + 1 < NCH:
            fire3(ci + 1, slots[(ci + 1) % 4])
        finish(ci, slots[ci % 4])

    dt, semt = slots[4]
    toff = NCH * C
    pltpu.make_async_copy(z_h.at[si.at[pl.ds(toff, CT)]], dt, semt).start()
    pltpu.make_async_copy(z_h.at[si.at[pl.ds(toff, CT)]], dt, semt).wait()
    pltpu.async_copy(rel_sp.at[ti.at[pl.ds(toff, CT)]], dt, semt, add=True)
    pltpu.make_async_copy(rel_sp.at[ti.at[pl.ds(toff, CT)]], dt, semt).wait()
    pltpu.async_copy(zn_h.at[di.at[pl.ds(toff, CT)]], dt, semt, add=True)
    pltpu.make_async_copy(zn_h.at[di.at[pl.ds(toff, CT)]], dt, semt).wait()
    lane_t = lax.iota(jnp.int32, 16)
    acc_t = jnp.zeros((16,), jnp.float32)
    for f in range(H):
        fv_t = (lane_t + f) & (H - 1)
        vd_t = plsc.load_gather(dt, [lane_t, fv_t])
        acc_t = acc_t + vd_t * vd_t
    ibits_t = plsc.bitcast(acc_t, jnp.int32)
    magic_t = jnp.full((16,), 0x5F3759DF, jnp.int32)
    y_t = plsc.bitcast(magic_t - (ibits_t >> 1), jnp.float32)
    for _ in range(3):
        y_t = y_t * (1.5 - 0.5 * acc_t * y_t * y_t)
    o[pl.ds(toff, 16)] = jnp.where(acc_t > 0.0, -(acc_t * y_t), 0.0)

    pltpu.sync_copy(o, out_h.at[pl.ds(base, EPW)])


def kernel(z, edge_index, edge_type, rel_emb):
    src = edge_index[0].astype(jnp.int32)
    dst = edge_index[1].astype(jnp.int32)
    et = edge_type.astype(jnp.int32)
    return _transe(z, -z, src, dst, et, rel_emb)
